# baseline (device time: 42093 ns/iter reference)
import jax
import jax.numpy as jnp
from jax import lax
from jax.experimental import pallas as pl
from jax.experimental.pallas import tpu as pltpu

B, SQ, H, D = 4, 256, 16, 64
HQ = H // 4
SCALE = D ** -0.5
MESH = pl.DeviceIdType.MESH


def _body(q_ref, kv_ref, out_ref, kvr_ref,
          y_s, y_r, xf_s, xf_r, zf_s, zf_r, xh_s, xh_r, zh_s, zh_r):
    x = lax.axis_index("x")
    y = lax.axis_index("y")
    z = lax.axis_index("z")
    ynbr = (x, 1 - y, z)
    xnbr = (1 - x, y, z)
    znbr = (x, y, 1 - z)
    qb = (2 * x + z) * HQ
    xqb = (2 * (1 - x) + z) * HQ
    zqb = (2 * x + (1 - z)) * HQ
    dqb = (2 * (1 - x) + (1 - z)) * HQ

    bar = pltpu.get_barrier_semaphore()
    for nbr in (ynbr, xnbr, znbr):
        pl.semaphore_signal(bar, inc=1, device_id=nbr, device_id_type=MESH)
    pl.semaphore_wait(bar, 3)

    y_rdmas = []
    for b in range(B):
        for kvi in range(2):
            r = pltpu.make_async_remote_copy(
                src_ref=kv_ref.at[kvi, b, pl.ds(qb, HQ)],
                dst_ref=kvr_ref.at[kvi, b, pl.ds(qb, HQ)],
                send_sem=y_s.at[2 * b + kvi], recv_sem=y_r.at[2 * b + kvi],
                device_id=ynbr, device_id_type=MESH)
            r.start()
            y_rdmas.append(r)

    def _fold(b, h):
        q = q_ref[b, h] * SCALE
        k = jnp.concatenate([kv_ref[0, b, h], kvr_ref[0, b, h]], axis=1)
        v = jnp.concatenate([kv_ref[1, b, h], kvr_ref[1, b, h]], axis=1)
        s_t = lax.dot_general(k, q, (((0,), (0,)), ((), ())),
                              preferred_element_type=jnp.float32)
        p = jnp.exp(s_t)
        l = jnp.sum(p, axis=0, keepdims=True)
        o = lax.dot_general(v, p.astype(jnp.bfloat16),
                            (((1,), (0,)), ((), ())),
                            preferred_element_type=jnp.float32)
        out_ref[b, h] = o * (1.0 / l)

    xf_rdmas, zf_rdmas = [], []
    for b in range(B):
        for kvi in range(2):
            i = 2 * b + kvi
            y_rdmas[i].wait_recv()
            for lst, sems, nbr in ((xf_rdmas, (xf_s, xf_r), xnbr),
                                   (zf_rdmas, (zf_s, zf_r), znbr)):
                r = pltpu.make_async_remote_copy(
                    src_ref=kvr_ref.at[kvi, b, pl.ds(qb, HQ)],
                    dst_ref=kvr_ref.at[kvi, b, pl.ds(qb, HQ)],
                    send_sem=sems[0].at[i], recv_sem=sems[1].at[i],
                    device_id=nbr, device_id_type=MESH)
                r.start()
                lst.append(r)
        for j in range(HQ):
            _fold(b, qb + j)

    xh_rdmas, zh_rdmas = [], []
    for b in range(B):
        xf_rdmas[2 * b].wait_recv()
        xf_rdmas[2 * b + 1].wait_recv()
        r = pltpu.make_async_remote_copy(
            src_ref=kvr_ref.at[:, b, pl.ds(xqb, HQ // 2)],
            dst_ref=kvr_ref.at[:, b, pl.ds(xqb, HQ // 2)],
            send_sem=zh_s.at[b], recv_sem=zh_r.at[b],
            device_id=znbr, device_id_type=MESH)
        r.start()
        zh_rdmas.append(r)

        zf_rdmas[2 * b].wait_recv()
        zf_rdmas[2 * b + 1].wait_recv()
        r = pltpu.make_async_remote_copy(
            src_ref=kvr_ref.at[:, b, pl.ds(zqb + HQ // 2, HQ // 2)],
            dst_ref=kvr_ref.at[:, b, pl.ds(zqb + HQ // 2, HQ // 2)],
            send_sem=xh_s.at[b], recv_sem=xh_r.at[b],
            device_id=xnbr, device_id_type=MESH)
        r.start()
        xh_rdmas.append(r)
        for j in range(HQ):
            _fold(b, xqb + j)
        for j in range(HQ):
            _fold(b, zqb + j)

    for b in range(B):
        zh_rdmas[b].wait_recv()
        xh_rdmas[b].wait_recv()
        for j in range(HQ):
            _fold(b, dqb + j)

    for r in y_rdmas + xf_rdmas + zf_rdmas + xh_rdmas + zh_rdmas:
        r.wait_send()


def kernel(Q, K, V):
    Qt = Q.astype(jnp.bfloat16).transpose(0, 2, 3, 1)
    Kt = K.astype(jnp.bfloat16).transpose(0, 2, 3, 1)
    Vt = V.astype(jnp.bfloat16).transpose(0, 2, 3, 1)
    kvt = jnp.stack([Kt, Vt])

    out = pl.pallas_call(
        _body,
        in_specs=[pl.BlockSpec(memory_space=pltpu.VMEM),
                  pl.BlockSpec(memory_space=pltpu.VMEM)],
        out_specs=pl.BlockSpec(memory_space=pltpu.VMEM),
        out_shape=jax.ShapeDtypeStruct((B, H, D, SQ), jnp.float32),
        scratch_shapes=[
            pltpu.VMEM((2, B, H, D, SQ), jnp.bfloat16),
        ] + [pltpu.SemaphoreType.DMA((2 * B,)) for _ in range(6)]
          + [pltpu.SemaphoreType.DMA((B,)) for _ in range(4)],
        compiler_params=pltpu.CompilerParams(collective_id=0),
    )(Qt, kvt)
    return out.transpose(0, 3, 1, 2)
